# baseline (speedup 1.0000x reference)
import functools

import jax
import jax.numpy as jnp
from jax import lax
from jax.experimental import pallas as pl
from jax.experimental.pallas import tpu as pltpu
from jax.experimental.pallas import tpu_sc as plsc

N = 10000
E = 160000
IN_DIM = 128
H = 64
EDGE_DIM = 4
NUM_LAYERS = 6
TW = 32

NC, NS = 2, 16
NWORK = NC * NS
CH = 128
NCHUNK = E // CH
NT_G = (NCHUNK + NWORK - 1) // NWORK
NPC = NCHUNK // NC
NT_S = (NPC + NS - 1) // NS
ROWS_PER_TILE = N // NS
ZCH = 125

BN = 1000
BE = 8000

def _b(x):
    return x.astype(jnp.bfloat16)


def _bdot(x, w):
    return jnp.dot(_b(x), _b(w), preferred_element_type=jnp.float32)


def _br(x):
    return _b(x).astype(jnp.float32)


def _table_block(h, rain_next, wi0, wi1, wi2):
    wsrc = jnp.concatenate([wi0[0:64], wi1[0:64], wi2[0:64]], axis=1)
    wdst = jnp.concatenate([wi0[64:128], wi1[64:128], wi2[64:128]], axis=1)
    wr = jnp.concatenate([wi0[136:137], wi1[136:137], wi2[136:137]], axis=1)
    ts = _bdot(h, wsrc) + _br(rain_next) * _br(wr)
    td = _bdot(h, wdst)
    z = jnp.zeros((h.shape[0], TW - 24), jnp.float32)
    return jnp.concatenate([ts, z], axis=1), jnp.concatenate([td, z], axis=1)


def _emb_body(x_ref, rain_ref, wemb_ref, wi0_ref, wi1_ref, wi2_ref,
              ts_ref, td_ref):
    h = _bdot(x_ref[...], wemb_ref[...])
    rain_next = rain_ref[:, 0:1]
    ts, td = _table_block(h, rain_next, wi0_ref[...], wi1_ref[...], wi2_ref[...])
    ts_ref[...] = ts
    td_ref[...] = td


def _emb_tables(inputs, rain0, W_emb, W_in0, W_in1, W_in2):
    return pl.pallas_call(
        _emb_body,
        grid=(N // BN,),
        in_specs=[
            pl.BlockSpec((BN, IN_DIM), lambda i: (i, 0)),
            pl.BlockSpec((BN, NUM_LAYERS), lambda i: (i, 0)),
            pl.BlockSpec((IN_DIM, H), lambda i: (0, 0)),
            pl.BlockSpec((137, 8), lambda i: (0, 0)),
            pl.BlockSpec((137, 8), lambda i: (0, 0)),
            pl.BlockSpec((137, 8), lambda i: (0, 0)),
        ],
        out_specs=(
            pl.BlockSpec((BN, TW), lambda i: (i, 0)),
            pl.BlockSpec((BN, TW), lambda i: (i, 0)),
        ),
        out_shape=(
            jax.ShapeDtypeStruct((N, TW), jnp.float32),
            jax.ShapeDtypeStruct((N, TW), jnp.float32),
        ),
    )(inputs, rain0, W_emb, W_in0, W_in1, W_in2)


def _node_body(layer, with_tables, agg_ref, rain_ref, won_ref, wrain_ref,
               wi0_ref, wi1_ref, wi2_ref, rout_ref, *table_refs):
    agg = agg_ref[0] + agg_ref[1]
    bias = rain_ref[:, layer:layer + 1]
    h = jnp.maximum(_bdot(agg, won_ref[...]) + bias, 0.0)
    rout_ref[...] = _bdot(h, wrain_ref[...])
    if with_tables:
        rain_next = rain_ref[:, layer + 1:layer + 2]
        ts, td = _table_block(h, rain_next,
                              wi0_ref[...], wi1_ref[...], wi2_ref[...])
        table_refs[0][...] = ts
        table_refs[1][...] = td


def _node_update(agg2, rain0, W_on, w_rain, W_in0, W_in1, W_in2, layer,
                 with_tables):
    out_shape = [jax.ShapeDtypeStruct((N, 1), jnp.float32)]
    out_specs = [pl.BlockSpec((BN, 1), lambda i: (i, 0))]
    if with_tables:
        out_shape += [jax.ShapeDtypeStruct((N, TW), jnp.float32)] * 2
        out_specs += [pl.BlockSpec((BN, TW), lambda i: (i, 0))] * 2
    return pl.pallas_call(
        functools.partial(_node_body, layer, with_tables),
        grid=(N // BN,),
        in_specs=[
            pl.BlockSpec((NC, BN, H), lambda i: (0, i, 0)),
            pl.BlockSpec((BN, NUM_LAYERS), lambda i: (i, 0)),
            pl.BlockSpec((H, H), lambda i: (0, 0)),
            pl.BlockSpec((H, 1), lambda i: (0, 0)),
            pl.BlockSpec((137, 8), lambda i: (0, 0)),
            pl.BlockSpec((137, 8), lambda i: (0, 0)),
            pl.BlockSpec((137, 8), lambda i: (0, 0)),
        ],
        out_specs=tuple(out_specs),
        out_shape=tuple(out_shape),
    )(agg2, rain0, W_on, w_rain, W_in0, W_in1, W_in2)


_SC_MESH = plsc.VectorSubcoreMesh(core_axis_name="c", subcore_axis_name="s",
                                  num_cores=NC, num_subcores=NS)


@functools.partial(
    pl.kernel,
    out_type=jax.ShapeDtypeStruct((E, TW), jnp.float32),
    mesh=_SC_MESH,
    compiler_params=pltpu.CompilerParams(use_tc_tiling_on_sc=False),
    scratch_types=[
        pltpu.VMEM((2, CH), jnp.int32),
        pltpu.VMEM((2, CH), jnp.int32),
        pltpu.VMEM((2, CH, TW), jnp.float32),
        pltpu.VMEM((2, CH, TW), jnp.float32),
        pltpu.VMEM_SHARED((N, TW), jnp.float32),
        pltpu.VMEM_SHARED((N, TW), jnp.float32),
        pltpu.SemaphoreType.DMA,
        pltpu.SemaphoreType.DMA,
        pltpu.SemaphoreType.DMA,
        pltpu.SemaphoreType.DMA,
        pltpu.SemaphoreType.DMA,
        pltpu.SemaphoreType.DMA,
    ],
)
def _gather_call(src_hbm, dst_hbm, ts_hbm, td_hbm, o_hbm,
                 si, di, b1, b2, tssp, tdsp,
                 semi0, semi1, semg0, semg1, semw0, semw1):
    sid = lax.axis_index("s")
    wid = sid * NC + lax.axis_index("c")
    semi = (semi0, semi1)
    semg = (semg0, semg1)
    semw = (semw0, semw1)

    pltpu.sync_copy(ts_hbm.at[pl.ds(sid * ROWS_PER_TILE, ROWS_PER_TILE)],
                    tssp.at[pl.ds(sid * ROWS_PER_TILE, ROWS_PER_TILE)])
    pltpu.sync_copy(td_hbm.at[pl.ds(sid * ROWS_PER_TILE, ROWS_PER_TILE)],
                    tdsp.at[pl.ds(sid * ROWS_PER_TILE, ROWS_PER_TILE)])
    plsc.subcore_barrier()

    def valid(t):
        t = jnp.asarray(t, jnp.int32)
        return jnp.logical_and(t >= 0, wid + t * NWORK < NCHUNK)

    def cbase(t):
        return (wid + t * NWORK) * CH

    def issue_idx(t, b):
        @pl.when(valid(t))
        def _():
            base = cbase(t)
            pltpu.async_copy(src_hbm.at[pl.ds(base, CH)], si.at[b], semi[b])
            pltpu.async_copy(dst_hbm.at[pl.ds(base, CH)], di.at[b], semi[b])

    def wait_idx(t, b):
        @pl.when(valid(t))
        def _():
            base = cbase(t)
            pltpu.make_async_copy(
                src_hbm.at[pl.ds(base, CH)], si.at[b], semi[b]).wait()
            pltpu.make_async_copy(
                dst_hbm.at[pl.ds(base, CH)], di.at[b], semi[b]).wait()

    def issue_gather(b):
        pltpu.async_copy(tssp.at[si.at[b]], b1.at[b], semg[b])
        pltpu.async_copy(tdsp.at[di.at[b]], b2.at[b], semg[b])

    def wait_gather(t, b):
        @pl.when(valid(t))
        def _():
            pltpu.make_async_copy(tssp.at[si.at[b]], b1.at[b], semg[b]).wait()
            pltpu.make_async_copy(tdsp.at[di.at[b]], b2.at[b], semg[b]).wait()

    def wait_wb(t, b):
        @pl.when(valid(t))
        def _():
            pltpu.make_async_copy(
                b1.at[b], o_hbm.at[pl.ds(cbase(t), CH)], semw[b]).wait()

    issue_idx(0, 0)
    issue_idx(1, 1)
    wait_idx(0, 0)

    @pl.when(valid(0))
    def _():
        issue_gather(0)

    def pair(tt, carry):
        for b in (0, 1):
            t_ = tt * 2 + b

            @pl.when(valid(t_))
            def _(t=t_, b=b):
                nb = 1 - b
                wait_wb(t - 1, nb)
                wait_idx(t + 1, nb)

                @pl.when(valid(t + 1))
                def _():
                    issue_gather(nb)

                wait_gather(t, b)
                issue_idx(t + 2, b)

                def add_row(i, c2):
                    for j in range(TW // 16):
                        sl = pl.ds(16 * j, 16)
                        b1[b, i, sl] = b1[b, i, sl] + b2[b, i, sl]
                    return c2

                lax.fori_loop(0, CH, add_row, 0, unroll=8)
                pltpu.async_copy(b1.at[b], o_hbm.at[pl.ds(cbase(t), CH)],
                                 semw[b])

        return carry

    lax.fori_loop(0, (NT_G + 1) // 2, pair, 0)
    wait_wb(NT_G - 1, (NT_G - 1) % 2)

    @pl.when(jnp.logical_not(valid(NT_G - 1)))
    def _():
        wait_wb(NT_G - 2, (NT_G - 2) % 2)


def _edge_body(s_ref, e24_ref, wi0_ref, wi1_ref, wi2_ref,
               woe_ref, wef0_ref, wsel12_ref, msg_ref):
    e24b = _b(e24_ref[...])
    ef = jnp.dot(e24b, _b(wef0_ref[...]), preferred_element_type=jnp.float32)
    e12 = jnp.dot(e24b, _b(wsel12_ref[...]), preferred_element_type=jnp.float32)
    s = s_ref[...]
    for j, wref in enumerate((wi0_ref, wi1_ref, wi2_ref)):
        w = wref[128:136, :]
        ef = jnp.maximum(s[:, 8 * j:8 * j + 8] + _bdot(ef, w), 0.0)
    woe = woe_ref[...]
    msg_ref[...] = jnp.maximum(
        _bdot(ef, woe[0:8]) + _bdot(e12, woe[8:20]), 0.0)


def _edge_msgs(s, e24, W_in0, W_in1, W_in2, W_oe, wef0, wsel12):
    return pl.pallas_call(
        _edge_body,
        grid=(E // BE,),
        in_specs=[
            pl.BlockSpec((BE, TW), lambda i: (i, 0)),
            pl.BlockSpec((BE, EDGE_DIM * NUM_LAYERS), lambda i: (i, 0)),
            pl.BlockSpec((137, 8), lambda i: (0, 0)),
            pl.BlockSpec((137, 8), lambda i: (0, 0)),
            pl.BlockSpec((137, 8), lambda i: (0, 0)),
            pl.BlockSpec((20, H), lambda i: (0, 0)),
            pl.BlockSpec((EDGE_DIM * NUM_LAYERS, 8), lambda i: (0, 0)),
            pl.BlockSpec((EDGE_DIM * NUM_LAYERS, 12), lambda i: (0, 0)),
        ],
        out_specs=pl.BlockSpec((BE, H), lambda i: (i, 0)),
        out_shape=jax.ShapeDtypeStruct((E, H), jnp.float32),
        compiler_params=pltpu.CompilerParams(vmem_limit_bytes=100 * 1024 * 1024),
    )(s, e24, W_in0, W_in1, W_in2, W_oe, wef0, wsel12)


@functools.partial(
    pl.kernel,
    out_type=jax.ShapeDtypeStruct((NC * N, H), jnp.float32),
    mesh=_SC_MESH,
    compiler_params=pltpu.CompilerParams(use_tc_tiling_on_sc=False),
    scratch_types=[
        pltpu.VMEM((2, CH), jnp.int32),
        pltpu.VMEM((2, CH, H), jnp.float32),
        pltpu.VMEM((ZCH, H), jnp.float32),
        pltpu.VMEM_SHARED((N, H), jnp.float32),
        pltpu.SemaphoreType.DMA,
        pltpu.SemaphoreType.DMA,
        pltpu.SemaphoreType.DMA,
        pltpu.SemaphoreType.DMA,
    ],
)
def _scatter_call(dst_hbm, msg_hbm, out_hbm, di, mb, zb, acc,
                  seml0, seml1, sems0, sems1):
    cid = lax.axis_index("c")
    sid = lax.axis_index("s")
    seml = (seml0, seml1)
    sems = (sems0, sems1)

    def zrow(i, carry):
        for j in range(H // 16):
            zb[i, pl.ds(16 * j, 16)] = jnp.zeros((16,), jnp.float32)
        return carry

    lax.fori_loop(0, ZCH, zrow, 0, unroll=4)
    for z in range(ROWS_PER_TILE // ZCH):
        pltpu.sync_copy(zb, acc.at[pl.ds(sid * ROWS_PER_TILE + z * ZCH, ZCH)])
    plsc.subcore_barrier()

    def valid(t):
        t = jnp.asarray(t, jnp.int32)
        return jnp.logical_and(t >= 0, sid + t * NS < NPC)

    def cbase(t):
        return (cid * NPC + sid + t * NS) * CH

    def issue_load(t, b):
        @pl.when(valid(t))
        def _():
            base = cbase(t)
            pltpu.async_copy(dst_hbm.at[pl.ds(base, CH)], di.at[b], seml[b])
            pltpu.async_copy(msg_hbm.at[pl.ds(base, CH)], mb.at[b], seml[b])

    def wait_load(t, b):
        @pl.when(valid(t))
        def _():
            base = cbase(t)
            pltpu.make_async_copy(
                dst_hbm.at[pl.ds(base, CH)], di.at[b], seml[b]).wait()
            pltpu.make_async_copy(
                msg_hbm.at[pl.ds(base, CH)], mb.at[b], seml[b]).wait()

    def wait_scat(t, b):
        @pl.when(valid(t))
        def _():
            pltpu.make_async_copy(mb.at[b], acc.at[di.at[b]], sems[b]).wait()

    issue_load(0, 0)

    def pair2(tt, carry):
        for b in (0, 1):
            t_ = tt * 2 + b

            @pl.when(valid(t_))
            def _(t=t_, b=b):
                wait_load(t, b)
                pltpu.async_copy(mb.at[b], acc.at[di.at[b]], sems[b],
                                 add=True)
                wait_scat(t - 1, 1 - b)
                issue_load(t + 1, 1 - b)

        return carry

    lax.fori_loop(0, (NT_S + 1) // 2, pair2, 0)
    wait_scat(NT_S - 1, (NT_S - 1) % 2)

    @pl.when(jnp.logical_not(valid(NT_S - 1)))
    def _():
        wait_scat(NT_S - 2, (NT_S - 2) % 2)

    plsc.subcore_barrier()

    pltpu.sync_copy(
        acc.at[pl.ds(sid * ROWS_PER_TILE, ROWS_PER_TILE)],
        out_hbm.at[pl.ds(cid * N + sid * ROWS_PER_TILE, ROWS_PER_TILE)])


def kernel(inputs, e_feats, rain0, edge_index, W_emb, W_in0, W_in1, W_in2,
           W_oe, W_on, w_rain):
    src = edge_index[0].astype(jnp.int32)
    dst = edge_index[1].astype(jnp.int32)
    e24 = e_feats.reshape(E, EDGE_DIM * NUM_LAYERS)

    eye4 = jnp.eye(EDGE_DIM, dtype=jnp.float32)
    wef0s, wsel12s = [], []
    for l in range(NUM_LAYERS):
        sel = jnp.zeros((EDGE_DIM * NUM_LAYERS, EDGE_DIM), jnp.float32)
        sel = sel.at[l::NUM_LAYERS, :].set(eye4)
        wef0s.append(jnp.concatenate([sel, sel], axis=1))
        wsel12s.append(jnp.concatenate([sel, sel, sel], axis=1))

    ts, td = _emb_tables(inputs, rain0, W_emb, W_in0, W_in1, W_in2)
    rains = []
    for l in range(NUM_LAYERS):
        s = _gather_call(src, dst, ts, td)
        msg = _edge_msgs(s, e24, W_in0, W_in1, W_in2, W_oe,
                         wef0s[l], wsel12s[l])
        agg = _scatter_call(dst, msg).reshape(NC, N, H)
        if l < NUM_LAYERS - 1:
            rout, ts, td = _node_update(agg, rain0, W_on, w_rain,
                                        W_in0, W_in1, W_in2, l, True)
        else:
            (rout,) = _node_update(agg, rain0, W_on, w_rain,
                                   W_in0, W_in1, W_in2, l, False)
        rains.append(rout)
    return jnp.concatenate(rains, axis=1)


# speedup vs baseline: 1.0070x; 1.0070x over previous
import functools

import jax
import jax.numpy as jnp
from jax import lax
from jax.experimental import pallas as pl
from jax.experimental.pallas import tpu as pltpu
from jax.experimental.pallas import tpu_sc as plsc

N = 10000
E = 160000
IN_DIM = 128
H = 64
EDGE_DIM = 4
NUM_LAYERS = 6
TW = 32

NC, NS = 2, 16
NWORK = NC * NS
CH = 128
NCHUNK = E // CH
NT_G = (NCHUNK + NWORK - 1) // NWORK
NPC = NCHUNK // NC
NT_S = (NPC + NS - 1) // NS
ROWS_PER_TILE = N // NS
ZCH = 125

BN = 1000
BE = 8000

def _b(x):
    return x.astype(jnp.bfloat16)


def _bdot(x, w):
    return jnp.dot(_b(x), _b(w), preferred_element_type=jnp.float32)


def _br(x):
    return _b(x).astype(jnp.float32)


def _table_block(h, rain_next, wi0, wi1, wi2):
    wsrc = jnp.concatenate([wi0[0:64], wi1[0:64], wi2[0:64]], axis=1)
    wdst = jnp.concatenate([wi0[64:128], wi1[64:128], wi2[64:128]], axis=1)
    wr = jnp.concatenate([wi0[136:137], wi1[136:137], wi2[136:137]], axis=1)
    ts = _bdot(h, wsrc) + _br(rain_next) * _br(wr)
    td = _bdot(h, wdst)
    z = jnp.zeros((h.shape[0], TW - 24), jnp.float32)
    return jnp.concatenate([ts, z], axis=1), jnp.concatenate([td, z], axis=1)


def _emb_body(x_ref, rain_ref, wemb_ref, wi0_ref, wi1_ref, wi2_ref,
              ts_ref, td_ref):
    h = _bdot(x_ref[...], wemb_ref[...])
    rain_next = rain_ref[:, 0:1]
    ts, td = _table_block(h, rain_next, wi0_ref[...], wi1_ref[...], wi2_ref[...])
    ts_ref[...] = ts
    td_ref[...] = td


def _emb_tables(inputs, rain0, W_emb, W_in0, W_in1, W_in2):
    return pl.pallas_call(
        _emb_body,
        grid=(N // BN,),
        in_specs=[
            pl.BlockSpec((BN, IN_DIM), lambda i: (i, 0)),
            pl.BlockSpec((BN, NUM_LAYERS), lambda i: (i, 0)),
            pl.BlockSpec((IN_DIM, H), lambda i: (0, 0)),
            pl.BlockSpec((137, 8), lambda i: (0, 0)),
            pl.BlockSpec((137, 8), lambda i: (0, 0)),
            pl.BlockSpec((137, 8), lambda i: (0, 0)),
        ],
        out_specs=(
            pl.BlockSpec((BN, TW), lambda i: (i, 0)),
            pl.BlockSpec((BN, TW), lambda i: (i, 0)),
        ),
        out_shape=(
            jax.ShapeDtypeStruct((N, TW), jnp.float32),
            jax.ShapeDtypeStruct((N, TW), jnp.float32),
        ),
    )(inputs, rain0, W_emb, W_in0, W_in1, W_in2)


def _node_body(layer, with_tables, agg_ref, rain_ref, won_ref, wrain_ref,
               wi0_ref, wi1_ref, wi2_ref, rout_ref, *table_refs):
    agg = agg_ref[0] + agg_ref[1]
    bias = rain_ref[:, layer:layer + 1]
    h = jnp.maximum(_bdot(agg, won_ref[...]) + bias, 0.0)
    rout_ref[...] = _bdot(h, wrain_ref[...])
    if with_tables:
        rain_next = rain_ref[:, layer + 1:layer + 2]
        ts, td = _table_block(h, rain_next,
                              wi0_ref[...], wi1_ref[...], wi2_ref[...])
        table_refs[0][...] = ts
        table_refs[1][...] = td


def _node_update(agg2, rain0, W_on, w_rain, W_in0, W_in1, W_in2, layer,
                 with_tables):
    out_shape = [jax.ShapeDtypeStruct((N, 1), jnp.float32)]
    out_specs = [pl.BlockSpec((BN, 1), lambda i: (i, 0))]
    if with_tables:
        out_shape += [jax.ShapeDtypeStruct((N, TW), jnp.float32)] * 2
        out_specs += [pl.BlockSpec((BN, TW), lambda i: (i, 0))] * 2
    return pl.pallas_call(
        functools.partial(_node_body, layer, with_tables),
        grid=(N // BN,),
        in_specs=[
            pl.BlockSpec((NC, BN, H), lambda i: (0, i, 0)),
            pl.BlockSpec((BN, NUM_LAYERS), lambda i: (i, 0)),
            pl.BlockSpec((H, H), lambda i: (0, 0)),
            pl.BlockSpec((H, 1), lambda i: (0, 0)),
            pl.BlockSpec((137, 8), lambda i: (0, 0)),
            pl.BlockSpec((137, 8), lambda i: (0, 0)),
            pl.BlockSpec((137, 8), lambda i: (0, 0)),
        ],
        out_specs=tuple(out_specs),
        out_shape=tuple(out_shape),
    )(agg2, rain0, W_on, w_rain, W_in0, W_in1, W_in2)


_SC_MESH = plsc.VectorSubcoreMesh(core_axis_name="c", subcore_axis_name="s",
                                  num_cores=NC, num_subcores=NS)


@functools.partial(
    pl.kernel,
    out_type=jax.ShapeDtypeStruct((E, TW), jnp.float32),
    mesh=_SC_MESH,
    compiler_params=pltpu.CompilerParams(use_tc_tiling_on_sc=False),
    scratch_types=[
        pltpu.VMEM((2, CH), jnp.int32),
        pltpu.VMEM((2, CH), jnp.int32),
        pltpu.VMEM((2, CH, TW), jnp.float32),
        pltpu.VMEM((2, CH, TW), jnp.float32),
        pltpu.SemaphoreType.DMA,
        pltpu.SemaphoreType.DMA,
        pltpu.SemaphoreType.DMA,
        pltpu.SemaphoreType.DMA,
        pltpu.SemaphoreType.DMA,
        pltpu.SemaphoreType.DMA,
    ],
)
def _gather_call(src_hbm, dst_hbm, ts_hbm, td_hbm, o_hbm,
                 si, di, b1, b2,
                 semi0, semi1, semg0, semg1, semw0, semw1):
    wid = lax.axis_index("s") * NC + lax.axis_index("c")
    semi = (semi0, semi1)
    semg = (semg0, semg1)
    semw = (semw0, semw1)

    def valid(t):
        t = jnp.asarray(t, jnp.int32)
        return jnp.logical_and(t >= 0, wid + t * NWORK < NCHUNK)

    def cbase(t):
        return (wid + t * NWORK) * CH

    def issue_idx(t, b):
        @pl.when(valid(t))
        def _():
            base = cbase(t)
            pltpu.async_copy(src_hbm.at[pl.ds(base, CH)], si.at[b], semi[b])
            pltpu.async_copy(dst_hbm.at[pl.ds(base, CH)], di.at[b], semi[b])

    def wait_idx(t, b):
        @pl.when(valid(t))
        def _():
            base = cbase(t)
            pltpu.make_async_copy(
                src_hbm.at[pl.ds(base, CH)], si.at[b], semi[b]).wait()
            pltpu.make_async_copy(
                dst_hbm.at[pl.ds(base, CH)], di.at[b], semi[b]).wait()

    def issue_gather(b):
        pltpu.async_copy(ts_hbm.at[si.at[b]], b1.at[b], semg[b])
        pltpu.async_copy(td_hbm.at[di.at[b]], b2.at[b], semg[b])

    def wait_gather(t, b):
        @pl.when(valid(t))
        def _():
            pltpu.make_async_copy(ts_hbm.at[si.at[b]], b1.at[b], semg[b]).wait()
            pltpu.make_async_copy(td_hbm.at[di.at[b]], b2.at[b], semg[b]).wait()

    def wait_wb(t, b):
        @pl.when(valid(t))
        def _():
            pltpu.make_async_copy(
                b1.at[b], o_hbm.at[pl.ds(cbase(t), CH)], semw[b]).wait()

    issue_idx(0, 0)
    issue_idx(1, 1)
    wait_idx(0, 0)

    @pl.when(valid(0))
    def _():
        issue_gather(0)

    def pair(tt, carry):
        for b in (0, 1):
            t_ = tt * 2 + b

            @pl.when(valid(t_))
            def _(t=t_, b=b):
                nb = 1 - b
                wait_wb(t - 1, nb)
                wait_idx(t + 1, nb)

                @pl.when(valid(t + 1))
                def _():
                    issue_gather(nb)

                wait_gather(t, b)
                issue_idx(t + 2, b)

                def add_row(i, c2):
                    for j in range(TW // 16):
                        sl = pl.ds(16 * j, 16)
                        b1[b, i, sl] = b1[b, i, sl] + b2[b, i, sl]
                    return c2

                lax.fori_loop(0, CH, add_row, 0, unroll=8)
                pltpu.async_copy(b1.at[b], o_hbm.at[pl.ds(cbase(t), CH)],
                                 semw[b])

        return carry

    lax.fori_loop(0, (NT_G + 1) // 2, pair, 0)
    wait_wb(NT_G - 1, (NT_G - 1) % 2)

    @pl.when(jnp.logical_not(valid(NT_G - 1)))
    def _():
        wait_wb(NT_G - 2, (NT_G - 2) % 2)


def _edge_body(s_ref, e24_ref, wi0_ref, wi1_ref, wi2_ref,
               woe_ref, wef0_ref, wsel12_ref, msg_ref):
    e24b = _b(e24_ref[...])
    ef = jnp.dot(e24b, _b(wef0_ref[...]), preferred_element_type=jnp.float32)
    e12 = jnp.dot(e24b, _b(wsel12_ref[...]), preferred_element_type=jnp.float32)
    s = s_ref[...]
    for j, wref in enumerate((wi0_ref, wi1_ref, wi2_ref)):
        w = wref[128:136, :]
        ef = jnp.maximum(s[:, 8 * j:8 * j + 8] + _bdot(ef, w), 0.0)
    woe = woe_ref[...]
    msg_ref[...] = jnp.maximum(
        _bdot(ef, woe[0:8]) + _bdot(e12, woe[8:20]), 0.0)


def _edge_msgs(s, e24, W_in0, W_in1, W_in2, W_oe, wef0, wsel12):
    return pl.pallas_call(
        _edge_body,
        grid=(E // BE,),
        in_specs=[
            pl.BlockSpec((BE, TW), lambda i: (i, 0)),
            pl.BlockSpec((BE, EDGE_DIM * NUM_LAYERS), lambda i: (i, 0)),
            pl.BlockSpec((137, 8), lambda i: (0, 0)),
            pl.BlockSpec((137, 8), lambda i: (0, 0)),
            pl.BlockSpec((137, 8), lambda i: (0, 0)),
            pl.BlockSpec((20, H), lambda i: (0, 0)),
            pl.BlockSpec((EDGE_DIM * NUM_LAYERS, 8), lambda i: (0, 0)),
            pl.BlockSpec((EDGE_DIM * NUM_LAYERS, 12), lambda i: (0, 0)),
        ],
        out_specs=pl.BlockSpec((BE, H), lambda i: (i, 0)),
        out_shape=jax.ShapeDtypeStruct((E, H), jnp.float32),
        compiler_params=pltpu.CompilerParams(vmem_limit_bytes=100 * 1024 * 1024),
    )(s, e24, W_in0, W_in1, W_in2, W_oe, wef0, wsel12)


@functools.partial(
    pl.kernel,
    out_type=jax.ShapeDtypeStruct((NC * N, H), jnp.float32),
    mesh=_SC_MESH,
    compiler_params=pltpu.CompilerParams(use_tc_tiling_on_sc=False),
    scratch_types=[
        pltpu.VMEM((2, CH), jnp.int32),
        pltpu.VMEM((2, CH, H), jnp.float32),
        pltpu.VMEM((ZCH, H), jnp.float32),
        pltpu.VMEM_SHARED((N, H), jnp.float32),
        pltpu.SemaphoreType.DMA,
        pltpu.SemaphoreType.DMA,
        pltpu.SemaphoreType.DMA,
        pltpu.SemaphoreType.DMA,
    ],
)
def _scatter_call(dst_hbm, msg_hbm, out_hbm, di, mb, zb, acc,
                  seml0, seml1, sems0, sems1):
    cid = lax.axis_index("c")
    sid = lax.axis_index("s")
    seml = (seml0, seml1)
    sems = (sems0, sems1)

    def zrow(i, carry):
        for j in range(H // 16):
            zb[i, pl.ds(16 * j, 16)] = jnp.zeros((16,), jnp.float32)
        return carry

    lax.fori_loop(0, ZCH, zrow, 0, unroll=4)
    for z in range(ROWS_PER_TILE // ZCH):
        pltpu.sync_copy(zb, acc.at[pl.ds(sid * ROWS_PER_TILE + z * ZCH, ZCH)])
    plsc.subcore_barrier()

    def valid(t):
        t = jnp.asarray(t, jnp.int32)
        return jnp.logical_and(t >= 0, sid + t * NS < NPC)

    def cbase(t):
        return (cid * NPC + sid + t * NS) * CH

    def issue_load(t, b):
        @pl.when(valid(t))
        def _():
            base = cbase(t)
            pltpu.async_copy(dst_hbm.at[pl.ds(base, CH)], di.at[b], seml[b])
            pltpu.async_copy(msg_hbm.at[pl.ds(base, CH)], mb.at[b], seml[b])

    def wait_load(t, b):
        @pl.when(valid(t))
        def _():
            base = cbase(t)
            pltpu.make_async_copy(
                dst_hbm.at[pl.ds(base, CH)], di.at[b], seml[b]).wait()
            pltpu.make_async_copy(
                msg_hbm.at[pl.ds(base, CH)], mb.at[b], seml[b]).wait()

    def wait_scat(t, b):
        @pl.when(valid(t))
        def _():
            pltpu.make_async_copy(mb.at[b], acc.at[di.at[b]], sems[b]).wait()

    issue_load(0, 0)

    def pair2(tt, carry):
        for b in (0, 1):
            t_ = tt * 2 + b

            @pl.when(valid(t_))
            def _(t=t_, b=b):
                wait_load(t, b)
                pltpu.async_copy(mb.at[b], acc.at[di.at[b]], sems[b],
                                 add=True)
                wait_scat(t - 1, 1 - b)
                issue_load(t + 1, 1 - b)

        return carry

    lax.fori_loop(0, (NT_S + 1) // 2, pair2, 0)
    wait_scat(NT_S - 1, (NT_S - 1) % 2)

    @pl.when(jnp.logical_not(valid(NT_S - 1)))
    def _():
        wait_scat(NT_S - 2, (NT_S - 2) % 2)

    plsc.subcore_barrier()

    pltpu.sync_copy(
        acc.at[pl.ds(sid * ROWS_PER_TILE, ROWS_PER_TILE)],
        out_hbm.at[pl.ds(cid * N + sid * ROWS_PER_TILE, ROWS_PER_TILE)])


def kernel(inputs, e_feats, rain0, edge_index, W_emb, W_in0, W_in1, W_in2,
           W_oe, W_on, w_rain):
    src = edge_index[0].astype(jnp.int32)
    dst = edge_index[1].astype(jnp.int32)
    e24 = e_feats.reshape(E, EDGE_DIM * NUM_LAYERS)

    eye4 = jnp.eye(EDGE_DIM, dtype=jnp.float32)
    wef0s, wsel12s = [], []
    for l in range(NUM_LAYERS):
        sel = jnp.zeros((EDGE_DIM * NUM_LAYERS, EDGE_DIM), jnp.float32)
        sel = sel.at[l::NUM_LAYERS, :].set(eye4)
        wef0s.append(jnp.concatenate([sel, sel], axis=1))
        wsel12s.append(jnp.concatenate([sel, sel, sel], axis=1))

    ts, td = _emb_tables(inputs, rain0, W_emb, W_in0, W_in1, W_in2)
    rains = []
    for l in range(NUM_LAYERS):
        s = _gather_call(src, dst, ts, td)
        msg = _edge_msgs(s, e24, W_in0, W_in1, W_in2, W_oe,
                         wef0s[l], wsel12s[l])
        agg = _scatter_call(dst, msg).reshape(NC, N, H)
        if l < NUM_LAYERS - 1:
            rout, ts, td = _node_update(agg, rain0, W_on, w_rain,
                                        W_in0, W_in1, W_in2, l, True)
        else:
            (rout,) = _node_update(agg, rain0, W_on, w_rain,
                                   W_in0, W_in1, W_in2, l, False)
        rains.append(rout)
    return jnp.concatenate(rains, axis=1)
